# Initial kernel scaffold; baseline (speedup 1.0000x reference)
import jax
import jax.numpy as jnp
from jax.experimental import pallas as pl


def kernel(x, emb, W1f, U1f, b1f, W1b, U1b, b1b, W2f, U2f, b2f, W2b, U2b, b2b, Wd, bd):
    raise NotImplementedError("write your pallas kernel here")


# speedup vs baseline: 1.6179x; 1.6179x over previous
import functools

import jax
import jax.numpy as jnp
from jax import lax
from jax.experimental import pallas as pl
from jax.experimental.pallas import tpu as pltpu
from jax.experimental.pallas import tpu_sc as plsc

B, T, V, D, U = 1024, 200, 100000, 128, 64
G = 4 * U
N = B * T


def _sc_gather(emb, idx3):
    info = plsc.get_sparse_core_info()
    nw = info.num_cores * info.num_subcores
    nch, ch = idx3.shape[1], idx3.shape[2]
    per_w = nch * ch
    mesh = plsc.VectorSubcoreMesh(core_axis_name="c", subcore_axis_name="s")

    @functools.partial(
        pl.kernel,
        mesh=mesh,
        compiler_params=pltpu.CompilerParams(use_tc_tiling_on_sc=False),
        out_type=jax.ShapeDtypeStruct((N, D), jnp.float32),
        scratch_types=[
            pltpu.VMEM((nch, ch), jnp.int32),
            pltpu.VMEM((ch, D), jnp.float32),
            pltpu.VMEM((ch, D), jnp.float32),
            pltpu.SemaphoreType.DMA,
            pltpu.SemaphoreType.DMA,
        ],
    )
    def k(emb_hbm, idx_hbm, out_hbm, idx_v, rows_a, rows_b, sem_a, sem_b):
        wid = lax.axis_index("s") * info.num_cores + lax.axis_index("c")
        base = pl.multiple_of(wid * per_w, 8)
        pltpu.sync_copy(idx_hbm.at[wid], idx_v)
        bufs = ((rows_a, sem_a), (rows_b, sem_b))

        def gather(c, buf, sem):
            return pltpu.async_copy(emb_hbm.at[idx_v.at[c]], buf, sem)

        gather(0, *bufs[0]).wait()
        for c in range(1, nch):
            cp = gather(c, *bufs[c % 2])
            prev = bufs[(c - 1) % 2][0]
            pltpu.sync_copy(
                prev, out_hbm.at[pl.ds(pl.multiple_of(base + (c - 1) * ch, 8), ch)]
            )
            cp.wait()
        pltpu.sync_copy(
            bufs[(nch - 1) % 2][0],
            out_hbm.at[pl.ds(pl.multiple_of(base + (nch - 1) * ch, 8), ch)],
        )

    return k(emb, idx3)


def _tanh(x):
    x = jnp.clip(x, -7.90531110763549805, 7.90531110763549805)
    x2 = x * x
    p = x2 * (-2.76076847742355e-16) + 2.00018790482477e-13
    p = x2 * p + (-8.60467152213735e-11)
    p = x2 * p + 5.12229709037114e-08
    p = x2 * p + 1.48572235717979e-05
    p = x2 * p + 6.37261928875436e-04
    p = x2 * p + 4.89352455891786e-03
    p = x * p
    q = x2 * 1.19825839466702e-06 + 1.18534705686654e-04
    q = x2 * q + 2.26843463243900e-03
    q = x2 * q + 4.89352518554385e-03
    r = 1.0 / q
    r = r * (2.0 - q * r)
    return p * r


def _sigmoid(x):
    return 0.5 + 0.5 * _tanh(0.5 * x)


def _rbf16(x):
    u = jax.lax.bitcast_convert_type(x, jnp.uint32)
    u = (u + jnp.uint32(0x7FFF) + ((u >> jnp.uint32(16)) & jnp.uint32(1)))
    u = u & jnp.uint32(0xFFFF0000)
    return jax.lax.bitcast_convert_type(u, jnp.float32)


def _bdot(a, b):
    return jnp.dot(_rbf16(a), b, preferred_element_type=jnp.float32)


def _cell(x, W_ref, U_ref, b_ref, h_ref, c_ref):
    z = _bdot(x, W_ref[...])
    z = z + _bdot(h_ref[...], U_ref[...])
    z = z + b_ref[...]
    i = _sigmoid(z[:, 0 * U:1 * U])
    f = _sigmoid(z[:, 1 * U:2 * U])
    g = _tanh(z[:, 2 * U:3 * U])
    o = _sigmoid(z[:, 3 * U:4 * U])
    c = f * c_ref[...] + i * g
    h = o * _tanh(c)
    c_ref[...] = c
    h_ref[...] = h
    return h


def _layer1_kernel(xf, xb, Wf, Uf, bf, Wb, Ub, bb, of, ob, hf, cf, hb, cb):
    t = pl.program_id(0)

    @pl.when(t == 0)
    def _():
        for r in (hf, cf, hb, cb):
            r[...] = jnp.zeros_like(r)

    of[0] = _cell(xf[0], Wf, Uf, bf, hf, cf)
    ob[0] = _cell(xb[0], Wb, Ub, bb, hb, cb)


def _layer1(x_tm, W1f, U1f, b1f, W1b, U1b, b1b):
    wspec = lambda s: pl.BlockSpec(s, lambda t: (0,) * len(s))
    return pl.pallas_call(
        _layer1_kernel,
        grid=(T,),
        in_specs=[
            pl.BlockSpec((1, B, D), lambda t: (t, 0, 0)),
            pl.BlockSpec((1, B, D), lambda t: (T - 1 - t, 0, 0)),
            wspec((D, G)), wspec((U, G)), wspec((1, G)),
            wspec((D, G)), wspec((U, G)), wspec((1, G)),
        ],
        out_specs=[
            pl.BlockSpec((1, B, U), lambda t: (t, 0, 0)),
            pl.BlockSpec((1, B, U), lambda t: (T - 1 - t, 0, 0)),
        ],
        out_shape=[jax.ShapeDtypeStruct((T, B, U), jnp.float32)] * 2,
        scratch_shapes=[pltpu.VMEM((B, U), jnp.float32)] * 4,
    )(x_tm, x_tm, W1f, U1f, b1f.reshape(1, G), W1b, U1b, b1b.reshape(1, G))


def _layer2_kernel(f1t, b1t, f1s, b1s,
                   Wfa, Wfb, Uf, bf, Wba, Wbb, Ub, bb,
                   Wdf, Wdb, bd, out, hf, cf, hb, cb, acc):
    t = pl.program_id(0)

    @pl.when(t == 0)
    def _():
        for r in (hf, cf, hb, cb, acc):
            r[...] = jnp.zeros_like(r)

    def cell2(xa, xb_, Wa, Wb_, U_ref, b_ref, h_ref, c_ref):
        z = _bdot(xa, Wa[...])
        z = z + _bdot(xb_, Wb_[...])
        z = z + _bdot(h_ref[...], U_ref[...])
        z = z + b_ref[...]
        i = _sigmoid(z[:, 0 * U:1 * U])
        f = _sigmoid(z[:, 1 * U:2 * U])
        g = _tanh(z[:, 2 * U:3 * U])
        o = _sigmoid(z[:, 3 * U:4 * U])
        c = f * c_ref[...] + i * g
        h = o * _tanh(c)
        c_ref[...] = c
        h_ref[...] = h
        return h

    h2f = cell2(f1t[0], b1t[0], Wfa, Wfb, Uf, bf, hf, cf)
    h2b = cell2(f1s[0], b1s[0], Wba, Wbb, Ub, bb, hb, cb)
    acc[...] = acc[...] + _bdot(h2f, Wdf[0]) + _bdot(h2b, Wdb[0])

    @pl.when(t == T - 1)
    def _():
        logits = acc[...] + bd[...]
        m = jnp.max(logits, axis=-1, keepdims=True)
        e = jnp.exp(logits - m)
        out[...] = e / jnp.sum(e, axis=-1, keepdims=True)


def _layer2(f1, b1, W2f, U2f, b2f, W2b, U2b, b2b, Wd, bd):
    wspec = lambda s: pl.BlockSpec(s, lambda t: (0,) * len(s))
    Wd3 = Wd.reshape(T, 2 * U, 3)
    return pl.pallas_call(
        _layer2_kernel,
        grid=(T,),
        in_specs=[
            pl.BlockSpec((1, B, U), lambda t: (t, 0, 0)),
            pl.BlockSpec((1, B, U), lambda t: (t, 0, 0)),
            pl.BlockSpec((1, B, U), lambda t: (T - 1 - t, 0, 0)),
            pl.BlockSpec((1, B, U), lambda t: (T - 1 - t, 0, 0)),
            wspec((U, G)), wspec((U, G)), wspec((U, G)), wspec((1, G)),
            wspec((U, G)), wspec((U, G)), wspec((U, G)), wspec((1, G)),
            pl.BlockSpec((1, U, 3), lambda t: (t, 0, 0)),
            pl.BlockSpec((1, U, 3), lambda t: (T - 1 - t, 0, 0)),
            wspec((1, 3)),
        ],
        out_specs=pl.BlockSpec((B, 3), lambda t: (0, 0)),
        out_shape=jax.ShapeDtypeStruct((B, 3), jnp.float32),
        scratch_shapes=[pltpu.VMEM((B, U), jnp.float32)] * 4
        + [pltpu.VMEM((B, 3), jnp.float32)],
    )(f1, b1, f1, b1,
      W2f[:U], W2f[U:], U2f, b2f.reshape(1, G),
      W2b[:U], W2b[U:], U2b, b2b.reshape(1, G),
      Wd3[:, :U, :], Wd3[:, U:, :], bd.reshape(1, 3))


def kernel(x, emb, W1f, U1f, b1f, W1b, U1b, b1b,
           W2f, U2f, b2f, W2b, U2b, b2b, Wd, bd):
    nw, ch = 32, 320
    nch = N // (nw * ch)
    idx3 = x.T.reshape(nw, nch, ch)
    rows = _sc_gather(emb, idx3)
    x_tm = rows.reshape(T, B, D)
    rb = lambda w: w.astype(jnp.bfloat16).astype(jnp.float32)
    f1, b1 = _layer1(x_tm, rb(W1f), rb(U1f), b1f, rb(W1b), rb(U1b), b1b)
    return _layer2(f1, b1, rb(W2f), rb(U2f), b2f, rb(W2b), rb(U2b), b2b,
                   rb(Wd), bd)


# speedup vs baseline: 4.0539x; 2.5057x over previous
import functools

import jax
import jax.numpy as jnp
from jax import lax
from jax.experimental import pallas as pl
from jax.experimental.pallas import tpu as pltpu
from jax.experimental.pallas import tpu_sc as plsc

B, T, V, D, U = 1024, 200, 100000, 128, 64
G = 4 * U
N = B * T


def _sc_gather(emb, idx3):
    info = plsc.get_sparse_core_info()
    nw = info.num_cores * info.num_subcores
    nch, ch = idx3.shape[1], idx3.shape[2]
    per_w = nch * ch
    mesh = plsc.VectorSubcoreMesh(core_axis_name="c", subcore_axis_name="s")

    @functools.partial(
        pl.kernel,
        mesh=mesh,
        compiler_params=pltpu.CompilerParams(use_tc_tiling_on_sc=False),
        out_type=jax.ShapeDtypeStruct((N, D), jnp.float32),
        scratch_types=[
            pltpu.VMEM((nch, ch), jnp.int32),
            pltpu.VMEM((ch, D), jnp.float32),
            pltpu.VMEM((ch, D), jnp.float32),
            pltpu.SemaphoreType.DMA,
            pltpu.SemaphoreType.DMA,
        ],
    )
    def k(emb_hbm, idx_hbm, out_hbm, idx_v, rows_a, rows_b, sem_a, sem_b):
        wid = lax.axis_index("s") * info.num_cores + lax.axis_index("c")
        base = pl.multiple_of(wid * per_w, 8)
        pltpu.sync_copy(idx_hbm.at[wid], idx_v)
        bufs = ((rows_a, sem_a), (rows_b, sem_b))

        def gather(c, buf, sem):
            return pltpu.async_copy(emb_hbm.at[idx_v.at[c]], buf, sem)

        gather(0, *bufs[0]).wait()
        for c in range(1, nch):
            cp = gather(c, *bufs[c % 2])
            prev = bufs[(c - 1) % 2][0]
            pltpu.sync_copy(
                prev, out_hbm.at[pl.ds(pl.multiple_of(base + (c - 1) * ch, 8), ch)]
            )
            cp.wait()
        pltpu.sync_copy(
            bufs[(nch - 1) % 2][0],
            out_hbm.at[pl.ds(pl.multiple_of(base + (nch - 1) * ch, 8), ch)],
        )

    return k(emb, idx3)


def _tanh(x):
    return jnp.tanh(x)


def _sigmoid(x):
    return jax.nn.sigmoid(x)


def _rbf16(x):
    u = jax.lax.bitcast_convert_type(x, jnp.uint32)
    u = (u + jnp.uint32(0x7FFF) + ((u >> jnp.uint32(16)) & jnp.uint32(1)))
    u = u & jnp.uint32(0xFFFF0000)
    return jax.lax.bitcast_convert_type(u, jnp.float32)


def _bdot(a, b):
    return jnp.dot(_rbf16(a), b, preferred_element_type=jnp.float32)


def _cell(x, W_ref, U_ref, b_ref, h_ref, c_ref):
    z = _bdot(x, W_ref[...])
    z = z + _bdot(h_ref[...], U_ref[...])
    z = z + b_ref[...]
    i = _sigmoid(z[:, 0 * U:1 * U])
    f = _sigmoid(z[:, 1 * U:2 * U])
    g = _tanh(z[:, 2 * U:3 * U])
    o = _sigmoid(z[:, 3 * U:4 * U])
    c = f * c_ref[...] + i * g
    h = o * _tanh(c)
    c_ref[...] = c
    h_ref[...] = h
    return h


def _layer1_kernel(xf, xb, Wf, Uf, bf, Wb, Ub, bb, of, ob, hf, cf, hb, cb):
    t = pl.program_id(0)

    @pl.when(t == 0)
    def _():
        for r in (hf, cf, hb, cb):
            r[...] = jnp.zeros_like(r)

    of[0] = _cell(xf[0], Wf, Uf, bf, hf, cf)
    ob[0] = _cell(xb[0], Wb, Ub, bb, hb, cb)


def _layer1(x_tm, W1f, U1f, b1f, W1b, U1b, b1b):
    wspec = lambda s: pl.BlockSpec(s, lambda t: (0,) * len(s))
    return pl.pallas_call(
        _layer1_kernel,
        grid=(T,),
        in_specs=[
            pl.BlockSpec((1, B, D), lambda t: (t, 0, 0)),
            pl.BlockSpec((1, B, D), lambda t: (T - 1 - t, 0, 0)),
            wspec((D, G)), wspec((U, G)), wspec((1, G)),
            wspec((D, G)), wspec((U, G)), wspec((1, G)),
        ],
        out_specs=[
            pl.BlockSpec((1, B, U), lambda t: (t, 0, 0)),
            pl.BlockSpec((1, B, U), lambda t: (T - 1 - t, 0, 0)),
        ],
        out_shape=[jax.ShapeDtypeStruct((T, B, U), jnp.float32)] * 2,
        scratch_shapes=[pltpu.VMEM((B, U), jnp.float32)] * 4,
    )(x_tm, x_tm, W1f, U1f, b1f.reshape(1, G), W1b, U1b, b1b.reshape(1, G))


def _layer2_kernel(f1t, b1t, f1s, b1s,
                   Wfa, Wfb, Uf, bf, Wba, Wbb, Ub, bb,
                   Wdf, Wdb, bd, out, hf, cf, hb, cb, acc):
    t = pl.program_id(0)

    @pl.when(t == 0)
    def _():
        for r in (hf, cf, hb, cb, acc):
            r[...] = jnp.zeros_like(r)

    def cell2(xa, xb_, Wa, Wb_, U_ref, b_ref, h_ref, c_ref):
        z = _bdot(xa, Wa[...])
        z = z + _bdot(xb_, Wb_[...])
        z = z + _bdot(h_ref[...], U_ref[...])
        z = z + b_ref[...]
        i = _sigmoid(z[:, 0 * U:1 * U])
        f = _sigmoid(z[:, 1 * U:2 * U])
        g = _tanh(z[:, 2 * U:3 * U])
        o = _sigmoid(z[:, 3 * U:4 * U])
        c = f * c_ref[...] + i * g
        h = o * _tanh(c)
        c_ref[...] = c
        h_ref[...] = h
        return h

    h2f = cell2(f1t[0], b1t[0], Wfa, Wfb, Uf, bf, hf, cf)
    h2b = cell2(f1s[0], b1s[0], Wba, Wbb, Ub, bb, hb, cb)
    acc[...] = acc[...] + _bdot(h2f, Wdf[0]) + _bdot(h2b, Wdb[0])

    @pl.when(t == T - 1)
    def _():
        logits = acc[...] + bd[...]
        m = jnp.max(logits, axis=-1, keepdims=True)
        e = jnp.exp(logits - m)
        out[...] = e / jnp.sum(e, axis=-1, keepdims=True)


def _layer2(f1, b1, W2f, U2f, b2f, W2b, U2b, b2b, Wd, bd):
    wspec = lambda s: pl.BlockSpec(s, lambda t: (0,) * len(s))
    Wd3 = Wd.reshape(T, 2 * U, 3)
    return pl.pallas_call(
        _layer2_kernel,
        grid=(T,),
        in_specs=[
            pl.BlockSpec((1, B, U), lambda t: (t, 0, 0)),
            pl.BlockSpec((1, B, U), lambda t: (t, 0, 0)),
            pl.BlockSpec((1, B, U), lambda t: (T - 1 - t, 0, 0)),
            pl.BlockSpec((1, B, U), lambda t: (T - 1 - t, 0, 0)),
            wspec((U, G)), wspec((U, G)), wspec((U, G)), wspec((1, G)),
            wspec((U, G)), wspec((U, G)), wspec((U, G)), wspec((1, G)),
            pl.BlockSpec((1, U, 3), lambda t: (t, 0, 0)),
            pl.BlockSpec((1, U, 3), lambda t: (T - 1 - t, 0, 0)),
            wspec((1, 3)),
        ],
        out_specs=pl.BlockSpec((B, 3), lambda t: (0, 0)),
        out_shape=jax.ShapeDtypeStruct((B, 3), jnp.float32),
        scratch_shapes=[pltpu.VMEM((B, U), jnp.float32)] * 4
        + [pltpu.VMEM((B, 3), jnp.float32)],
    )(f1, b1, f1, b1,
      W2f[:U], W2f[U:], U2f, b2f.reshape(1, G),
      W2b[:U], W2b[U:], U2b, b2b.reshape(1, G),
      Wd3[:, :U, :], Wd3[:, U:, :], bd.reshape(1, 3))


def kernel(x, emb, W1f, U1f, b1f, W1b, U1b, b1b,
           W2f, U2f, b2f, W2b, U2b, b2b, Wd, bd):
    nw, ch = 32, 320
    nch = N // (nw * ch)
    idx3 = x.T.reshape(nw, nch, ch)
    rows = _sc_gather(emb, idx3)
    x_tm = rows.reshape(T, B, D)
    rb = lambda w: w.astype(jnp.bfloat16).astype(jnp.float32)
    f1, b1 = _layer1(x_tm, rb(W1f), rb(U1f), b1f, rb(W1b), rb(U1b), b1b)
    return _layer2(f1, b1, rb(W2f), rb(U2f), b2f, rb(W2b), rb(U2b), b2b,
                   rb(Wd), bd)
